# baseline (speedup 1.0000x reference)
import functools

import jax
import jax.numpy as jnp
from jax import lax
from jax.experimental import pallas as pl
from jax.experimental.pallas import tpu as pltpu
from jax.experimental.pallas import tpu_sc as plsc

_NUM_AGENTS = 4096
_FEAT = 3


def _body(table_hbm, out_hbm):
    pltpu.sync_copy(table_hbm.at[pl.ds(0, _NUM_AGENTS), :], out_hbm)


_sc = functools.partial(
    pl.kernel,
    out_type=jax.ShapeDtypeStruct((_NUM_AGENTS, _FEAT), jnp.float32),
    mesh=plsc.ScalarSubcoreMesh(axis_name="c", num_cores=1),
)(_body)


def kernel(pos_phi, num_agents):
    return _sc(pos_phi)


# speedup vs baseline: 3.3484x; 3.3484x over previous
import functools

import jax
import jax.numpy as jnp
from jax import lax
from jax.experimental import pallas as pl
from jax.experimental.pallas import tpu as pltpu
from jax.experimental.pallas import tpu_sc as plsc

_NUM_AGENTS = 4096
_FEAT = 3

_INFO = plsc.get_sparse_core_info()
_NC = _INFO.num_cores
_NS = _INFO.num_subcores
_NW = _NC * _NS
_ROWS = _NUM_AGENTS // _NW


def _body(table_hbm, out_hbm, buf):
    wid = lax.axis_index("s") * _NC + lax.axis_index("c")
    r0 = wid * _ROWS
    pltpu.sync_copy(table_hbm.at[pl.ds(r0, _ROWS), :], buf)
    pltpu.sync_copy(buf, out_hbm.at[pl.ds(r0, _ROWS), :])


_sc = functools.partial(
    pl.kernel,
    out_type=jax.ShapeDtypeStruct((_NUM_AGENTS, _FEAT), jnp.float32),
    mesh=plsc.VectorSubcoreMesh(core_axis_name="c", subcore_axis_name="s"),
    scratch_types=[pltpu.VMEM((_ROWS, _FEAT), jnp.float32)],
)(_body)


def kernel(pos_phi, num_agents):
    return _sc(pos_phi)


# speedup vs baseline: 3.4176x; 1.0206x over previous
import functools

import jax
import jax.numpy as jnp
from jax import lax
from jax.experimental import pallas as pl
from jax.experimental.pallas import tpu as pltpu
from jax.experimental.pallas import tpu_sc as plsc

_NUM_AGENTS = 4096
_FEAT = 3

_NS = plsc.get_sparse_core_info().num_subcores
_ROWS = _NUM_AGENTS // _NS
_HALF = _ROWS // 2


def _body(table_hbm, out_hbm, b1, b2, s1, s2):
    sid = lax.axis_index("s")
    r0 = sid * _ROWS
    in1 = pltpu.make_async_copy(table_hbm.at[pl.ds(r0, _HALF), :], b1, s1)
    in2 = pltpu.make_async_copy(
        table_hbm.at[pl.ds(r0 + _HALF, _HALF), :], b2, s2
    )
    in1.start()
    in2.start()
    in1.wait()
    out1 = pltpu.make_async_copy(b1, out_hbm.at[pl.ds(r0, _HALF), :], s1)
    out1.start()
    in2.wait()
    out2 = pltpu.make_async_copy(
        b2, out_hbm.at[pl.ds(r0 + _HALF, _HALF), :], s2
    )
    out2.start()
    out1.wait()
    out2.wait()


_sc = functools.partial(
    pl.kernel,
    out_type=jax.ShapeDtypeStruct((_NUM_AGENTS, _FEAT), jnp.float32),
    mesh=plsc.VectorSubcoreMesh(
        core_axis_name="c", subcore_axis_name="s", num_cores=1
    ),
    scratch_types=[
        pltpu.VMEM((_HALF, _FEAT), jnp.float32),
        pltpu.VMEM((_HALF, _FEAT), jnp.float32),
        pltpu.SemaphoreType.DMA,
        pltpu.SemaphoreType.DMA,
    ],
)(_body)


def kernel(pos_phi, num_agents):
    return _sc(pos_phi)
